# baseline (speedup 1.0000x reference)
import functools

import jax
import jax.numpy as jnp
from jax import lax
from jax.experimental import pallas as pl
from jax.experimental.pallas import tpu as pltpu
from jax.experimental.pallas import tpu_sc as plsc

_B = 4096
_NUM = 13
_T = 26
_V = 100000
_D = 32

_NC = 2
_NS = 16
_NW = _NC * _NS
_L = 16
_BPW = _B // _NW

_BM = 512


@functools.cache
def _make_sc_gather():
    mesh = plsc.VectorSubcoreMesh(core_axis_name="c", subcore_axis_name="s")
    return pl.kernel(
        _sc_gather_body,
        mesh=mesh,
        out_type=jax.ShapeDtypeStruct((_B, _T * _D), jnp.float32),
        scratch_types=[
            pltpu.VMEM((_T, _BPW), jnp.int32),
            pltpu.VMEM((2, _BPW, _D), jnp.float32),
            pltpu.SemaphoreType.DMA,
            pltpu.SemaphoreType.DMA,
        ],
        compiler_params=pltpu.CompilerParams(use_tc_tiling_on_sc=False),
    )


def _sc_gather_body(tables_hbm, cat_hbm, out_hbm, idx_v, rows_v, sem0, sem1):
    wid = lax.axis_index("s") * _NC + lax.axis_index("c")
    b0 = wid * _BPW

    pltpu.sync_copy(cat_hbm.at[:, pl.ds(b0, _BPW)], idx_v)

    for t in range(_T):
        off = t * _V
        for j in range(_BPW // _L):
            sl = pl.ds(j * _L, _L)
            idx_v[t, sl] = idx_v[t, sl] + off

    sems = (sem0, sem1)
    copies = [None, None]
    copies[0] = pltpu.async_copy(tables_hbm.at[idx_v.at[0]], rows_v.at[0], sems[0])
    for t in range(_T):
        buf = t & 1
        nxt = (t + 1) & 1
        if t + 1 < _T:
            copies[nxt] = pltpu.async_copy(
                tables_hbm.at[idx_v.at[t + 1]], rows_v.at[nxt], sems[nxt]
            )
        copies[buf].wait()
        pltpu.sync_copy(
            rows_v.at[buf], out_hbm.at[pl.ds(b0, _BPW), pl.ds(t * _D, _D)]
        )


def _tc_body(num_ref, emb_ref, wW_ref, wb_ref, w1n_ref, w1e_ref, b1_ref,
             w2_ref, b2_ref, w3_ref, b3_ref, w4_ref, b4_ref, out_ref):
    f32 = jnp.float32
    num = num_ref[...]
    emb = emb_ref[...]

    h = jnp.dot(emb, w1e_ref[...], preferred_element_type=f32)
    h = h + jnp.dot(num, w1n_ref[...], preferred_element_type=f32)
    h = jnp.maximum(h + b1_ref[...], 0.0)
    h = jnp.maximum(jnp.dot(h, w2_ref[...], preferred_element_type=f32) + b2_ref[...], 0.0)
    h = jnp.maximum(jnp.dot(h, w3_ref[...], preferred_element_type=f32) + b3_ref[...], 0.0)
    dnn = jnp.dot(h, w4_ref[...], preferred_element_type=f32) + b4_ref[...]

    wide = jnp.maximum(jnp.dot(num, wW_ref[...], preferred_element_type=f32) + wb_ref[...], 0.0)

    r = lax.broadcasted_iota(jnp.int32, (_T * _D, _D), 0)
    c = lax.broadcasted_iota(jnp.int32, (_T * _D, _D), 1)
    sel = (r % _D == c).astype(f32)
    cat_sum = jnp.dot(emb, sel, preferred_element_type=f32)

    logit = dnn + cat_sum + wide
    out_ref[...] = jax.nn.sigmoid(logit)


def _tc_forward(num, emb, wW, wb, w1n, w1e, b1, w2, b2, w3, b3, w4, b4):
    h1 = w1e.shape[1]
    h2 = w2.shape[1]
    h3 = w3.shape[1]
    h4 = w4.shape[1]

    def row_map(i):
        return (i, 0)

    def fix_map(i):
        return (0, 0)

    return pl.pallas_call(
        _tc_body,
        grid=(_B // _BM,),
        in_specs=[
            pl.BlockSpec((_BM, _NUM), row_map),
            pl.BlockSpec((_BM, _T * _D), row_map),
            pl.BlockSpec((_NUM, 1), fix_map),
            pl.BlockSpec((1, 1), fix_map),
            pl.BlockSpec((_NUM, h1), fix_map),
            pl.BlockSpec((_T * _D, h1), fix_map),
            pl.BlockSpec((1, h1), fix_map),
            pl.BlockSpec((h1, h2), fix_map),
            pl.BlockSpec((1, h2), fix_map),
            pl.BlockSpec((h2, h3), fix_map),
            pl.BlockSpec((1, h3), fix_map),
            pl.BlockSpec((h3, h4), fix_map),
            pl.BlockSpec((1, h4), fix_map),
        ],
        out_specs=pl.BlockSpec((_BM, _D), row_map),
        out_shape=jax.ShapeDtypeStruct((_B, _D), jnp.float32),
    )(num, emb, wW, wb, w1n, w1e, b1, w2, b2, w3, b3, w4, b4)


def kernel(numerical_features, cat_features, emb_tables, wide_W, wide_b,
           deep_Ws, deep_bs):
    tables_flat = emb_tables.reshape(_T * _V, _D)
    emb_flat = _make_sc_gather()(tables_flat, cat_features)
    w1 = deep_Ws[0]
    return _tc_forward(
        numerical_features, emb_flat,
        wide_W, wide_b.reshape(1, 1),
        w1[:_NUM], w1[_NUM:], deep_bs[0].reshape(1, -1),
        deep_Ws[1], deep_bs[1].reshape(1, -1),
        deep_Ws[2], deep_bs[2].reshape(1, -1),
        deep_Ws[3], deep_bs[3].reshape(1, -1),
    )
